# Initial kernel scaffold; baseline (speedup 1.0000x reference)
import jax
import jax.numpy as jnp
from jax.experimental import pallas as pl


def kernel(x, edge_index, edge_attr, band_ids, stage_ids, params):
    raise NotImplementedError("write your pallas kernel here")


# speedup vs baseline: 11.4984x; 11.4984x over previous
import functools
import math

import jax
import jax.numpy as jnp
from jax import lax
from jax.experimental import pallas as pl
from jax.experimental.pallas import tpu as pltpu
from jax.experimental.pallas import tpu_sc as plsc

_NC = 2
_NS = 16
_NW = _NC * _NS
_L = 16
_CHUNK = 128

_H = 8
_DK = 16
_D = 128


def _dot(a, b):
    return jax.lax.dot_general(a, b, (((1,), (0,)), ((), ())),
                               precision=jax.lax.Precision.HIGHEST,
                               preferred_element_type=jnp.float32)


def _ln(x, g, b, eps=1e-5):
    mu = jnp.mean(x, axis=-1, keepdims=True)
    var = jnp.mean((x - mu) ** 2, axis=-1, keepdims=True)
    return (x - mu) / jnp.sqrt(var + eps) * g + b


def _erf(x):
    s = jnp.sign(x)
    ax = jnp.abs(x)
    t = 1.0 / (1.0 + 0.3275911 * ax)
    poly = t * (0.254829592 + t * (-0.284496736 + t * (1.421413741
               + t * (-1.453152027 + t * 1.061405429))))
    return s * (1.0 - poly * jnp.exp(-ax * ax))


def _gelu(x):
    return 0.5 * x * (1.0 + _erf(x * (1.0 / math.sqrt(2.0))))


def _in_proj_body(x_ref, w_ref, b_ref, oh_ref, emb_ref, o_ref):
    o_ref[...] = (_dot(x_ref[...], w_ref[...]) + b_ref[...]
                  + _dot(oh_ref[...], emb_ref[...]))


def _qkv_body(h_ref, wq_ref, wk_ref, wv_ref, q_ref, k_ref, v_ref):
    h = h_ref[...]
    q_ref[...] = _dot(h, wq_ref[...])
    k_ref[...] = _dot(h, wk_ref[...])
    v_ref[...] = _dot(h, wv_ref[...])


def _node_body(agg_ref, h_ref, wo_ref, bo_ref, g1_ref, b1n_ref,
               w1_ref, b1_ref, w2_ref, b2_ref, g2_ref, b2n_ref, o_ref):
    u = _dot(agg_ref[...], wo_ref[...]) + bo_ref[...] + h_ref[...]
    u = _ln(u, g1_ref[...], b1n_ref[...])
    ff = _dot(_gelu(_dot(u, w1_ref[...]) + b1_ref[...]), w2_ref[...]) + b2_ref[...]
    o_ref[...] = _ln(u + ff, g2_ref[...], b2n_ref[...])


def _out_body(h_ref, w_ref, b_ref, o_ref):
    o_ref[...] = _dot(h_ref[...], w_ref[...]) + b_ref[...]


def _row_spec(blk, d):
    return pl.BlockSpec((blk, d), lambda i: (i, 0))


def _full_spec(a, b):
    return pl.BlockSpec((a, b), lambda i: (0, 0))


def _tc_in_proj(x, w, b, onehot, emb):
    n, d_in = x.shape
    blk = 1000
    return pl.pallas_call(
        _in_proj_body,
        grid=(n // blk,),
        in_specs=[_row_spec(blk, d_in), _full_spec(d_in, _D),
                  _full_spec(1, _D), _row_spec(blk, _D), _full_spec(_D, _D)],
        out_specs=_row_spec(blk, _D),
        out_shape=jax.ShapeDtypeStruct((n, _D), jnp.float32),
    )(x, w, b, onehot, emb)


def _tc_qkv(h, wq, wk, wv):
    n = h.shape[0]
    blk = 1000
    sds = jax.ShapeDtypeStruct((n, _D), jnp.float32)
    return pl.pallas_call(
        _qkv_body,
        grid=(n // blk,),
        in_specs=[_row_spec(blk, _D)] + [_full_spec(_D, _D)] * 3,
        out_specs=[_row_spec(blk, _D)] * 3,
        out_shape=[sds, sds, sds],
    )(h, wq, wk, wv)


def _tc_node(agg, h, lp):
    n = h.shape[0]
    blk = 1000
    return pl.pallas_call(
        _node_body,
        grid=(n // blk,),
        in_specs=[_row_spec(blk, _D), _row_spec(blk, _D),
                  _full_spec(_D, _D), _full_spec(1, _D),
                  _full_spec(1, _D), _full_spec(1, _D),
                  _full_spec(_D, 4 * _D), _full_spec(1, 4 * _D),
                  _full_spec(4 * _D, _D), _full_spec(1, _D),
                  _full_spec(1, _D), _full_spec(1, _D)],
        out_specs=_row_spec(blk, _D),
        out_shape=jax.ShapeDtypeStruct((n, _D), jnp.float32),
    )(agg, h, lp['wo'], lp['bo'].reshape(1, _D),
      lp['ln1_g'].reshape(1, _D), lp['ln1_b'].reshape(1, _D),
      lp['w1'], lp['b1'].reshape(1, 4 * _D), lp['w2'],
      lp['b2'].reshape(1, _D), lp['ln2_g'].reshape(1, _D),
      lp['ln2_b'].reshape(1, _D))


def _tc_out(h, w, b):
    n = h.shape[0]
    blk = 1000
    return pl.pallas_call(
        _out_body,
        grid=(n // blk,),
        in_specs=[_row_spec(blk, _D), _full_spec(_D, _D), _full_spec(1, _D)],
        out_specs=_row_spec(blk, _D),
        out_shape=jax.ShapeDtypeStruct((n, _D), jnp.float32),
    )(h, w, b.reshape(1, _D))


def _sload(ref, i):
    return ref[pl.ds(i, _L)][0]


def _sc_edge_body(q_hbm, k_hbm, v_hbm, src_hbm, dst_hbm, band_hbm, attr_hbm,
                  sege_hbm, meta_hbm, btbl_hbm, out_hbm,
                  meta_v, btbl_v, srci, dsti, dstpad, bandi, attri, segei,
                  qrows, krows, vrows, stage, zbuf, semq, semk, semv):
    wid = lax.axis_index("s") * _NC + lax.axis_index("c")

    pltpu.sync_copy(meta_hbm, meta_v)
    pltpu.sync_copy(btbl_hbm, btbl_v)

    e_start = meta_v[0, pl.ds(wid, _L)][0]
    e_end = meta_v[1, pl.ds(wid, _L)][0]
    n_start = meta_v[2, pl.ds(wid, _L)][0]
    n_end = meta_v[3, pl.ds(wid, _L)][0]

    iota = lax.iota(jnp.int32, _L)
    iota16 = iota * _DK
    iota_h = jnp.minimum(iota, _H - 1)
    lane_lt8 = iota < _H
    zero16 = jnp.zeros((_L,), jnp.int32)
    neg_inf = jnp.full((_L,), -jnp.inf, jnp.float32)
    zerof = jnp.zeros((_L,), jnp.float32)

    def _zb(i, _):
        for j in range(_D // _L):
            zbuf[i, pl.ds(j * _L, _L)] = zerof
        return 0
    lax.fori_loop(0, zbuf.shape[0], _zb, 0)

    ncnt = n_end - n_start
    zb = zbuf.shape[0]
    nblocks = ncnt // zb

    def _zero_block(i, _):
        pltpu.sync_copy(zbuf, out_hbm.at[pl.ds(n_start + i * zb, zb)])
        return 0
    lax.fori_loop(0, nblocks, _zero_block, 0)

    @pl.when(ncnt >= zb)
    def _():
        pltpu.sync_copy(zbuf, out_hbm.at[pl.ds(n_end - zb, zb)])

    @pl.when(jnp.logical_and(ncnt < zb, ncnt > 0))
    def _():
        def _zr(i, _):
            pltpu.sync_copy(zbuf.at[pl.ds(0, 1)],
                            out_hbm.at[pl.ds(n_start + i, 1)])
            return 0
        lax.fori_loop(0, ncnt, _zr, 0)

    a0 = (e_start // 8) * 8
    nch = (e_end - a0 + _CHUNK - 1) // _CHUNK

    def _edge_body(e, carry):
        m = carry[0]
        s = carry[1]
        acc = carry[2:]
        attr = _sload(attri, e)
        band = _sload(bandi, e)
        se = _sload(segei, e)
        node = _sload(dstpad, e)
        row = jnp.full((_L,), e, jnp.int32)

        dot = zerof
        for d in range(_DK):
            col = iota16 + d
            qd = plsc.load_gather(qrows, [row, col])
            kd = plsc.load_gather(krows, [row, col])
            dot = dot + qd * kd
        bias = plsc.load_gather(btbl_v, [jnp.full((_L,), band, jnp.int32),
                                         iota_h])
        a = (dot * 0.25 + bias) * attr
        mn = jnp.maximum(m, a)
        c = jnp.exp(m - mn)
        ev = jnp.exp(a - mn)
        s2 = s * c + ev
        acc2 = []
        for d in range(_DK):
            col = iota16 + d
            vd = plsc.load_gather(vrows, [row, col])
            acc2.append(acc[d] * c + ev * vd)

        @pl.when(se == 1)
        def _():
            rcp = 1.0 / (s2 + 1e-16)
            for d in range(_DK):
                plsc.store_scatter(stage, [zero16, iota16 + d],
                                   acc2[d] * rcp, mask=lane_lt8)
            pltpu.sync_copy(stage, out_hbm.at[pl.ds(node, 1)])

        keep = se == 0
        m_o = jnp.where(keep, mn, neg_inf)
        s_o = jnp.where(keep, s2, zerof)
        acc_o = tuple(jnp.where(keep, a2, zerof) for a2 in acc2)
        return (m_o, s_o) + acc_o

    def _chunk_body(ci, carry):
        base = a0 + ci * _CHUNK
        pltpu.sync_copy(src_hbm.at[pl.ds(base, _CHUNK)], srci)
        pltpu.sync_copy(dst_hbm.at[pl.ds(base, _CHUNK)], dsti)
        pltpu.sync_copy(dst_hbm.at[pl.ds(base, _CHUNK + _L)], dstpad)
        pltpu.sync_copy(band_hbm.at[pl.ds(base, _CHUNK + _L)], bandi)
        pltpu.sync_copy(attr_hbm.at[pl.ds(base, _CHUNK + _L)], attri)
        pltpu.sync_copy(sege_hbm.at[pl.ds(base, _CHUNK + _L)], segei)
        dq = pltpu.async_copy(q_hbm.at[dsti], qrows, semq)
        dk = pltpu.async_copy(k_hbm.at[srci], krows, semk)
        dv = pltpu.async_copy(v_hbm.at[srci], vrows, semv)
        dq.wait()
        dk.wait()
        dv.wait()
        lo = jnp.maximum(e_start - base, 0)
        hi = jnp.minimum(e_end - base, _CHUNK)
        return lax.fori_loop(lo, hi, _edge_body, carry)

    init = (neg_inf, zerof) + tuple(zerof for _ in range(_DK))
    lax.fori_loop(0, nch, _chunk_body, init)


def _sc_edge(q, k, v, srcp, dstp, bandp, attrp, segep, meta, btbl):
    n = q.shape[0]
    mesh = plsc.VectorSubcoreMesh(core_axis_name="c", subcore_axis_name="s")
    f = pl.kernel(
        _sc_edge_body,
        out_type=jax.ShapeDtypeStruct((n, _D), jnp.float32),
        mesh=mesh,
        compiler_params=pltpu.CompilerParams(use_tc_tiling_on_sc=False,
                                             needs_layout_passes=False),
        scratch_types=[
            pltpu.VMEM((4, _NW + _L), jnp.int32),
            pltpu.VMEM((6, _H), jnp.float32),
            pltpu.VMEM((_CHUNK,), jnp.int32),
            pltpu.VMEM((_CHUNK,), jnp.int32),
            pltpu.VMEM((_CHUNK + _L,), jnp.int32),
            pltpu.VMEM((_CHUNK + _L,), jnp.int32),
            pltpu.VMEM((_CHUNK + _L,), jnp.float32),
            pltpu.VMEM((_CHUNK + _L,), jnp.int32),
            pltpu.VMEM((_CHUNK, _D), jnp.float32),
            pltpu.VMEM((_CHUNK, _D), jnp.float32),
            pltpu.VMEM((_CHUNK, _D), jnp.float32),
            pltpu.VMEM((1, _D), jnp.float32),
            pltpu.VMEM((64, _D), jnp.float32),
            pltpu.SemaphoreType.DMA,
            pltpu.SemaphoreType.DMA,
            pltpu.SemaphoreType.DMA,
        ],
    )
    return f(q, k, v, srcp, dstp, bandp, attrp, segep, meta, btbl)


def kernel(x, edge_index, edge_attr, band_ids, stage_ids, params):
    n = x.shape[0]
    e = edge_index.shape[1]

    src = edge_index[0].astype(jnp.int32)
    dst = edge_index[1].astype(jnp.int32)

    perm = jnp.argsort(dst)
    dst_s = dst[perm]
    src_s = src[perm]
    band_s = band_ids[perm].astype(jnp.int32)
    attr_s = edge_attr[perm]
    seg_end = jnp.concatenate(
        [(dst_s[1:] != dst_s[:-1]), jnp.ones((1,), jnp.bool_)]).astype(jnp.int32)

    row_ptr = jnp.searchsorted(dst_s, jnp.arange(n + 1, dtype=jnp.int32),
                               side='left').astype(jnp.int32)
    targets = (jnp.arange(_NW + 1, dtype=jnp.int32) * e) // _NW
    ncut = jnp.searchsorted(row_ptr, targets, side='left').astype(jnp.int32)
    ncut = ncut.at[-1].set(n)
    e_cut = row_ptr[ncut]
    meta = jnp.stack([e_cut[:-1], e_cut[1:], ncut[:-1], ncut[1:]]).astype(jnp.int32)
    meta = jnp.pad(meta, ((0, 0), (0, _L)))

    pad = 2 * _CHUNK
    srcp = jnp.pad(src_s, (0, pad))
    dstp = jnp.pad(dst_s, (0, pad))
    bandp = jnp.pad(band_s, (0, pad))
    attrp = jnp.pad(attr_s, (0, pad))
    segep = jnp.pad(seg_end, (0, pad))

    onehot = (stage_ids[:, None] == jnp.arange(_D)[None, :]).astype(jnp.float32)
    emb = jnp.zeros((_D, _D), jnp.float32).at[:params['stage_emb'].shape[0]].set(
        params['stage_emb'])

    h = _tc_in_proj(x, params['in_w'], params['in_b'].reshape(1, _D),
                    onehot, emb)

    for lp in params['layers']:
        q, k, v = _tc_qkv(h, lp['wq'], lp['wk'], lp['wv'])
        btbl = jnp.zeros((6, _H), jnp.float32).at[:5].set(lp['band_bias'])
        agg = _sc_edge(q, k, v, srcp, dstp, bandp, attrp, segep, meta, btbl)
        h = _tc_node(agg, h, lp)

    return _tc_out(h, params['out_w'], params['out_b'])


# speedup vs baseline: 17.7036x; 1.5397x over previous
import functools
import math

import jax
import jax.numpy as jnp
from jax import lax
from jax.experimental import pallas as pl
from jax.experimental.pallas import tpu as pltpu
from jax.experimental.pallas import tpu_sc as plsc

_NC = 2
_NS = 16
_NW = _NC * _NS
_L = 16
_CHUNK = 128

_H = 8
_DK = 16
_D = 128


def _dot(a, b):
    return jax.lax.dot_general(a, b, (((1,), (0,)), ((), ())),
                               precision=jax.lax.Precision.HIGHEST,
                               preferred_element_type=jnp.float32)


def _ln(x, g, b, eps=1e-5):
    mu = jnp.mean(x, axis=-1, keepdims=True)
    var = jnp.mean((x - mu) ** 2, axis=-1, keepdims=True)
    return (x - mu) / jnp.sqrt(var + eps) * g + b


def _erf(x):
    s = jnp.sign(x)
    ax = jnp.abs(x)
    t = 1.0 / (1.0 + 0.3275911 * ax)
    poly = t * (0.254829592 + t * (-0.284496736 + t * (1.421413741
               + t * (-1.453152027 + t * 1.061405429))))
    return s * (1.0 - poly * jnp.exp(-ax * ax))


def _gelu(x):
    return 0.5 * x * (1.0 + _erf(x * (1.0 / math.sqrt(2.0))))


def _in_proj_body(x_ref, w_ref, b_ref, oh_ref, emb_ref, o_ref):
    o_ref[...] = (_dot(x_ref[...], w_ref[...]) + b_ref[...]
                  + _dot(oh_ref[...], emb_ref[...]))


def _qkv_body(h_ref, wq_ref, wk_ref, wv_ref, q_ref, k_ref, v_ref):
    h = h_ref[...]
    q_ref[...] = _dot(h, wq_ref[...])
    k_ref[...] = _dot(h, wk_ref[...])
    v_ref[...] = _dot(h, wv_ref[...])


def _node_body(agg_ref, h_ref, wo_ref, bo_ref, g1_ref, b1n_ref,
               w1_ref, b1_ref, w2_ref, b2_ref, g2_ref, b2n_ref, o_ref):
    u = _dot(agg_ref[...], wo_ref[...]) + bo_ref[...] + h_ref[...]
    u = _ln(u, g1_ref[...], b1n_ref[...])
    ff = _dot(_gelu(_dot(u, w1_ref[...]) + b1_ref[...]), w2_ref[...]) + b2_ref[...]
    o_ref[...] = _ln(u + ff, g2_ref[...], b2n_ref[...])


def _out_body(h_ref, w_ref, b_ref, o_ref):
    o_ref[...] = _dot(h_ref[...], w_ref[...]) + b_ref[...]


def _row_spec(blk, d):
    return pl.BlockSpec((blk, d), lambda i: (i, 0))


def _full_spec(a, b):
    return pl.BlockSpec((a, b), lambda i: (0, 0))


def _tc_in_proj(x, w, b, onehot, emb):
    n, d_in = x.shape
    blk = 1000
    return pl.pallas_call(
        _in_proj_body,
        grid=(n // blk,),
        in_specs=[_row_spec(blk, d_in), _full_spec(d_in, _D),
                  _full_spec(1, _D), _row_spec(blk, _D), _full_spec(_D, _D)],
        out_specs=_row_spec(blk, _D),
        out_shape=jax.ShapeDtypeStruct((n, _D), jnp.float32),
    )(x, w, b, onehot, emb)


def _tc_qkv(h, wq, wk, wv):
    n = h.shape[0]
    blk = 1000
    sds = jax.ShapeDtypeStruct((n, _D), jnp.float32)
    return pl.pallas_call(
        _qkv_body,
        grid=(n // blk,),
        in_specs=[_row_spec(blk, _D)] + [_full_spec(_D, _D)] * 3,
        out_specs=[_row_spec(blk, _D)] * 3,
        out_shape=[sds, sds, sds],
    )(h, wq, wk, wv)


def _tc_node(agg, h, lp):
    n = h.shape[0]
    blk = 1000
    return pl.pallas_call(
        _node_body,
        grid=(n // blk,),
        in_specs=[_row_spec(blk, _D), _row_spec(blk, _D),
                  _full_spec(_D, _D), _full_spec(1, _D),
                  _full_spec(1, _D), _full_spec(1, _D),
                  _full_spec(_D, 4 * _D), _full_spec(1, 4 * _D),
                  _full_spec(4 * _D, _D), _full_spec(1, _D),
                  _full_spec(1, _D), _full_spec(1, _D)],
        out_specs=_row_spec(blk, _D),
        out_shape=jax.ShapeDtypeStruct((n, _D), jnp.float32),
    )(agg, h, lp['wo'], lp['bo'].reshape(1, _D),
      lp['ln1_g'].reshape(1, _D), lp['ln1_b'].reshape(1, _D),
      lp['w1'], lp['b1'].reshape(1, 4 * _D), lp['w2'],
      lp['b2'].reshape(1, _D), lp['ln2_g'].reshape(1, _D),
      lp['ln2_b'].reshape(1, _D))


def _tc_out(h, w, b):
    n = h.shape[0]
    blk = 1000
    return pl.pallas_call(
        _out_body,
        grid=(n // blk,),
        in_specs=[_row_spec(blk, _D), _full_spec(_D, _D), _full_spec(1, _D)],
        out_specs=_row_spec(blk, _D),
        out_shape=jax.ShapeDtypeStruct((n, _D), jnp.float32),
    )(h, w, b.reshape(1, _D))


def _sload(ref, i):
    return ref[pl.ds(i, _L)][0]


def _sc_edge_body(q_hbm, k_hbm, v_hbm, src_hbm, dst_hbm, band_hbm, attr_hbm,
                  sege_hbm, meta_hbm, btbl_hbm, out_hbm,
                  meta_v, btbl_v, srci, dsti, dstpad, bandi, attri, segei,
                  qrows, krows, vrows, stage, zbuf, semq, semk, semv):
    wid = lax.axis_index("s") * _NC + lax.axis_index("c")

    pltpu.sync_copy(meta_hbm, meta_v)
    pltpu.sync_copy(btbl_hbm, btbl_v)

    e_start = meta_v[0, pl.ds(wid, _L)][0]
    e_end = meta_v[1, pl.ds(wid, _L)][0]
    n_start = meta_v[2, pl.ds(wid, _L)][0]
    n_end = meta_v[3, pl.ds(wid, _L)][0]

    iota = lax.iota(jnp.int32, _L)
    iota16 = iota * _DK
    iota_h = jnp.minimum(iota, _H - 1)
    lane_lt8 = iota < _H
    zero16 = jnp.zeros((_L,), jnp.int32)
    neg_inf = jnp.full((_L,), -jnp.inf, jnp.float32)
    zerof = jnp.zeros((_L,), jnp.float32)

    def _zb(i, _):
        for j in range(_D // _L):
            zbuf[i, pl.ds(j * _L, _L)] = zerof
        return 0
    lax.fori_loop(0, zbuf.shape[0], _zb, 0)

    ncnt = n_end - n_start
    zb = zbuf.shape[0]
    nblocks = ncnt // zb

    def _zero_block(i, _):
        pltpu.sync_copy(zbuf, out_hbm.at[pl.ds(n_start + i * zb, zb)])
        return 0
    lax.fori_loop(0, nblocks, _zero_block, 0)

    @pl.when(ncnt >= zb)
    def _():
        pltpu.sync_copy(zbuf, out_hbm.at[pl.ds(n_end - zb, zb)])

    @pl.when(jnp.logical_and(ncnt < zb, ncnt > 0))
    def _():
        def _zr(i, _):
            pltpu.sync_copy(zbuf.at[pl.ds(0, 1)],
                            out_hbm.at[pl.ds(n_start + i, 1)])
            return 0
        lax.fori_loop(0, ncnt, _zr, 0)

    a0 = (e_start // 8) * 8
    nch = (e_end - a0 + _CHUNK - 1) // _CHUNK

    def _edge_body(e, carry):
        m = carry[0]
        s = carry[1]
        acc = carry[2:]
        attr = _sload(attri, e)
        band = _sload(bandi, e)
        se = _sload(segei, e)
        node = _sload(dstpad, e)
        row = jnp.full((_L,), e, jnp.int32)

        dot = zerof
        for d in range(_DK):
            col = iota16 + d
            qd = plsc.load_gather(qrows, [row, col])
            kd = plsc.load_gather(krows, [row, col])
            dot = dot + qd * kd
        bias = plsc.load_gather(btbl_v, [jnp.full((_L,), band, jnp.int32),
                                         iota_h])
        a = (dot * 0.25 + bias) * attr
        mn = jnp.maximum(m, a)
        c = jnp.exp(m - mn)
        ev = jnp.exp(a - mn)
        s2 = s * c + ev
        acc2 = []
        for d in range(_DK):
            col = iota16 + d
            vd = plsc.load_gather(vrows, [row, col])
            acc2.append(acc[d] * c + ev * vd)

        @pl.when(se == 1)
        def _():
            rcp = 1.0 / (s2 + 1e-16)
            for d in range(_DK):
                plsc.store_scatter(stage, [zero16, iota16 + d],
                                   acc2[d] * rcp, mask=lane_lt8)
            pltpu.sync_copy(stage, out_hbm.at[pl.ds(node, 1)])

        keep = se == 0
        m_o = jnp.where(keep, mn, neg_inf)
        s_o = jnp.where(keep, s2, zerof)
        acc_o = tuple(jnp.where(keep, a2, zerof) for a2 in acc2)
        return (m_o, s_o) + acc_o

    def _chunk_body(ci, carry):
        base = a0 + ci * _CHUNK
        pltpu.sync_copy(src_hbm.at[pl.ds(base, _CHUNK)], srci)
        pltpu.sync_copy(dst_hbm.at[pl.ds(base, _CHUNK)], dsti)
        pltpu.sync_copy(dst_hbm.at[pl.ds(base, _CHUNK + _L)], dstpad)
        pltpu.sync_copy(band_hbm.at[pl.ds(base, _CHUNK + _L)], bandi)
        pltpu.sync_copy(attr_hbm.at[pl.ds(base, _CHUNK + _L)], attri)
        pltpu.sync_copy(sege_hbm.at[pl.ds(base, _CHUNK + _L)], segei)
        dq = pltpu.async_copy(q_hbm.at[dsti], qrows, semq)
        dk = pltpu.async_copy(k_hbm.at[srci], krows, semk)
        dv = pltpu.async_copy(v_hbm.at[srci], vrows, semv)
        dq.wait()
        dk.wait()
        dv.wait()
        lo = jnp.maximum(e_start - base, 0)
        hi = jnp.minimum(e_end - base, _CHUNK)
        return lax.fori_loop(lo, hi, _edge_body, carry)

    init = (neg_inf, zerof) + tuple(zerof for _ in range(_DK))
    lax.fori_loop(0, nch, _chunk_body, init)


def _sc_edge(q, k, v, srcp, dstp, bandp, attrp, segep, meta, btbl):
    n = q.shape[0]
    mesh = plsc.VectorSubcoreMesh(core_axis_name="c", subcore_axis_name="s")
    f = pl.kernel(
        _sc_edge_body,
        out_type=jax.ShapeDtypeStruct((n, _D), jnp.float32),
        mesh=mesh,
        compiler_params=pltpu.CompilerParams(use_tc_tiling_on_sc=False,
                                             needs_layout_passes=False),
        scratch_types=[
            pltpu.VMEM((4, _NW + _L), jnp.int32),
            pltpu.VMEM((6, _H), jnp.float32),
            pltpu.VMEM((_CHUNK,), jnp.int32),
            pltpu.VMEM((_CHUNK,), jnp.int32),
            pltpu.VMEM((_CHUNK + _L,), jnp.int32),
            pltpu.VMEM((_CHUNK + _L,), jnp.int32),
            pltpu.VMEM((_CHUNK + _L,), jnp.float32),
            pltpu.VMEM((_CHUNK + _L,), jnp.int32),
            pltpu.VMEM((_CHUNK, _D), jnp.float32),
            pltpu.VMEM((_CHUNK, _D), jnp.float32),
            pltpu.VMEM((_CHUNK, _D), jnp.float32),
            pltpu.VMEM((1, _D), jnp.float32),
            pltpu.VMEM((64, _D), jnp.float32),
            pltpu.SemaphoreType.DMA,
            pltpu.SemaphoreType.DMA,
            pltpu.SemaphoreType.DMA,
        ],
    )
    return f(q, k, v, srcp, dstp, bandp, attrp, segep, meta, btbl)


def kernel(x, edge_index, edge_attr, band_ids, stage_ids, params):
    n = x.shape[0]
    e = edge_index.shape[1]

    src = edge_index[0].astype(jnp.int32)
    dst = edge_index[1].astype(jnp.int32)

    perm = jnp.argsort(dst)
    dst_s = dst[perm]
    src_s = src[perm]
    band_s = band_ids[perm].astype(jnp.int32)
    attr_s = edge_attr[perm]
    seg_end = jnp.concatenate(
        [(dst_s[1:] != dst_s[:-1]), jnp.ones((1,), jnp.bool_)]).astype(jnp.int32)

    tpos = (jnp.arange(1, _NW, dtype=jnp.int32) * e) // _NW
    node_at = dst_s[tpos]
    ecut_in = jnp.searchsorted(dst_s, node_at, side='left').astype(jnp.int32)
    e_cut = jnp.concatenate([jnp.zeros((1,), jnp.int32), ecut_in,
                             jnp.full((1,), e, jnp.int32)])
    ncut = jnp.concatenate([jnp.zeros((1,), jnp.int32), node_at,
                            jnp.full((1,), n, jnp.int32)])
    meta = jnp.stack([e_cut[:-1], e_cut[1:], ncut[:-1], ncut[1:]]).astype(jnp.int32)
    meta = jnp.pad(meta, ((0, 0), (0, _L)))

    pad = 2 * _CHUNK
    srcp = jnp.pad(src_s, (0, pad))
    dstp = jnp.pad(dst_s, (0, pad))
    bandp = jnp.pad(band_s, (0, pad))
    attrp = jnp.pad(attr_s, (0, pad))
    segep = jnp.pad(seg_end, (0, pad))

    onehot = (stage_ids[:, None] == jnp.arange(_D)[None, :]).astype(jnp.float32)
    emb = jnp.zeros((_D, _D), jnp.float32).at[:params['stage_emb'].shape[0]].set(
        params['stage_emb'])

    h = _tc_in_proj(x, params['in_w'], params['in_b'].reshape(1, _D),
                    onehot, emb)

    for lp in params['layers']:
        q, k, v = _tc_qkv(h, lp['wq'], lp['wk'], lp['wv'])
        btbl = jnp.zeros((6, _H), jnp.float32).at[:5].set(lp['band_bias'])
        agg = _sc_edge(q, k, v, srcp, dstp, bandp, attrp, segep, meta, btbl)
        h = _tc_node(agg, h, lp)

    return _tc_out(h, params['out_w'], params['out_b'])


# speedup vs baseline: 20.3446x; 1.1492x over previous
import functools
import math

import jax
import jax.numpy as jnp
from jax import lax
from jax.experimental import pallas as pl
from jax.experimental.pallas import tpu as pltpu
from jax.experimental.pallas import tpu_sc as plsc

_NC = 2
_NS = 16
_NW = _NC * _NS
_L = 16
_CHUNK = 128

_H = 8
_DK = 16
_D = 128


def _dot(a, b):
    return jax.lax.dot_general(a, b, (((1,), (0,)), ((), ())),
                               precision=jax.lax.Precision.HIGHEST,
                               preferred_element_type=jnp.float32)


def _ln(x, g, b, eps=1e-5):
    mu = jnp.mean(x, axis=-1, keepdims=True)
    var = jnp.mean((x - mu) ** 2, axis=-1, keepdims=True)
    return (x - mu) / jnp.sqrt(var + eps) * g + b


def _erf(x):
    s = jnp.sign(x)
    ax = jnp.abs(x)
    t = 1.0 / (1.0 + 0.3275911 * ax)
    poly = t * (0.254829592 + t * (-0.284496736 + t * (1.421413741
               + t * (-1.453152027 + t * 1.061405429))))
    return s * (1.0 - poly * jnp.exp(-ax * ax))


def _gelu(x):
    return 0.5 * x * (1.0 + _erf(x * (1.0 / math.sqrt(2.0))))


def _in_proj_body(x_ref, w_ref, b_ref, oh_ref, emb_ref, o_ref):
    o_ref[...] = (_dot(x_ref[...], w_ref[...]) + b_ref[...]
                  + _dot(oh_ref[...], emb_ref[...]))


def _qkv_body(h_ref, wq_ref, wk_ref, wv_ref, q_ref, k_ref, v_ref):
    h = h_ref[...]
    q_ref[...] = _dot(h, wq_ref[...])
    k_ref[...] = _dot(h, wk_ref[...])
    v_ref[...] = _dot(h, wv_ref[...])


def _node_body(agg_ref, h_ref, wo_ref, bo_ref, g1_ref, b1n_ref,
               w1_ref, b1_ref, w2_ref, b2_ref, g2_ref, b2n_ref, o_ref):
    u = _dot(agg_ref[...], wo_ref[...]) + bo_ref[...] + h_ref[...]
    u = _ln(u, g1_ref[...], b1n_ref[...])
    ff = _dot(_gelu(_dot(u, w1_ref[...]) + b1_ref[...]), w2_ref[...]) + b2_ref[...]
    o_ref[...] = _ln(u + ff, g2_ref[...], b2n_ref[...])


def _out_body(h_ref, w_ref, b_ref, o_ref):
    o_ref[...] = _dot(h_ref[...], w_ref[...]) + b_ref[...]


def _row_spec(blk, d):
    return pl.BlockSpec((blk, d), lambda i: (i, 0))


def _full_spec(a, b):
    return pl.BlockSpec((a, b), lambda i: (0, 0))


def _tc_in_proj(x, w, b, onehot, emb):
    n, d_in = x.shape
    blk = 1000
    return pl.pallas_call(
        _in_proj_body,
        grid=(n // blk,),
        in_specs=[_row_spec(blk, d_in), _full_spec(d_in, _D),
                  _full_spec(1, _D), _row_spec(blk, _D), _full_spec(_D, _D)],
        out_specs=_row_spec(blk, _D),
        out_shape=jax.ShapeDtypeStruct((n, _D), jnp.float32),
    )(x, w, b, onehot, emb)


def _tc_qkv(h, wq, wk, wv):
    n = h.shape[0]
    blk = 1000
    sds = jax.ShapeDtypeStruct((n, _D), jnp.float32)
    return pl.pallas_call(
        _qkv_body,
        grid=(n // blk,),
        in_specs=[_row_spec(blk, _D)] + [_full_spec(_D, _D)] * 3,
        out_specs=[_row_spec(blk, _D)] * 3,
        out_shape=[sds, sds, sds],
    )(h, wq, wk, wv)


def _tc_node(agg, h, lp):
    n = h.shape[0]
    blk = 1000
    return pl.pallas_call(
        _node_body,
        grid=(n // blk,),
        in_specs=[_row_spec(blk, _D), _row_spec(blk, _D),
                  _full_spec(_D, _D), _full_spec(1, _D),
                  _full_spec(1, _D), _full_spec(1, _D),
                  _full_spec(_D, 4 * _D), _full_spec(1, 4 * _D),
                  _full_spec(4 * _D, _D), _full_spec(1, _D),
                  _full_spec(1, _D), _full_spec(1, _D)],
        out_specs=_row_spec(blk, _D),
        out_shape=jax.ShapeDtypeStruct((n, _D), jnp.float32),
    )(agg, h, lp['wo'], lp['bo'].reshape(1, _D),
      lp['ln1_g'].reshape(1, _D), lp['ln1_b'].reshape(1, _D),
      lp['w1'], lp['b1'].reshape(1, 4 * _D), lp['w2'],
      lp['b2'].reshape(1, _D), lp['ln2_g'].reshape(1, _D),
      lp['ln2_b'].reshape(1, _D))


def _tc_out(h, w, b):
    n = h.shape[0]
    blk = 1000
    return pl.pallas_call(
        _out_body,
        grid=(n // blk,),
        in_specs=[_row_spec(blk, _D), _full_spec(_D, _D), _full_spec(1, _D)],
        out_specs=_row_spec(blk, _D),
        out_shape=jax.ShapeDtypeStruct((n, _D), jnp.float32),
    )(h, w, b.reshape(1, _D))


_FLUSH = 64


def _sc_edge_body(q_hbm, k_hbm, v_hbm, src_hbm, dst_hbm, edata_hbm,
                  meta_hbm, btbl_hbm, out_hbm,
                  meta_v, btbl_v, srci, dsti, edata, qrows, krows, vrows,
                  outbuf, oidx, zbuf, semq, semk, semv, semo):
    wid = lax.axis_index("s") * _NC + lax.axis_index("c")
    pad_row = out_hbm.shape[0] - 1

    pltpu.sync_copy(meta_hbm, meta_v)
    pltpu.sync_copy(btbl_hbm, btbl_v)

    e_start = meta_v[0, pl.ds(wid, _L)][0]
    e_end = meta_v[1, pl.ds(wid, _L)][0]
    n_start = meta_v[2, pl.ds(wid, _L)][0]
    n_end = meta_v[3, pl.ds(wid, _L)][0]

    iota = lax.iota(jnp.int32, _L)
    iota16 = iota * _DK
    iota_h = jnp.minimum(iota, _H - 1)
    lane_lt8 = iota < _H
    lane0 = iota < 1
    neg_inf = jnp.full((_L,), -jnp.inf, jnp.float32)
    zerof = jnp.zeros((_L,), jnp.float32)
    padv = jnp.full((_L,), pad_row, jnp.int32)

    zb = zbuf.shape[0]
    def _zb(i, _):
        for j in range(_D // _L):
            zbuf[i, pl.ds(j * _L, _L)] = zerof
        return 0
    lax.fori_loop(0, zb, _zb, 0)
    for j in range(_FLUSH // _L):
        oidx[pl.ds(j * _L, _L)] = padv

    ncnt = n_end - n_start
    nblocks = ncnt // zb

    def _zero_block(i, _):
        pltpu.sync_copy(zbuf, out_hbm.at[pl.ds(n_start + i * zb, zb)])
        return 0
    lax.fori_loop(0, nblocks, _zero_block, 0)

    @pl.when(ncnt >= zb)
    def _():
        pltpu.sync_copy(zbuf, out_hbm.at[pl.ds(n_end - zb, zb)])

    @pl.when(jnp.logical_and(ncnt < zb, ncnt > 0))
    def _():
        def _zr(i, _):
            pltpu.sync_copy(zbuf.at[pl.ds(0, 1)],
                            out_hbm.at[pl.ds(n_start + i, 1)])
            return 0
        lax.fori_loop(0, ncnt, _zr, 0)

    a0 = (e_start // 8) * 8
    nch = (e_end - a0 + _CHUNK - 1) // _CHUNK

    bufs = ((srci[0], dsti[0], edata[0], qrows[0], krows[0], vrows[0],
             semq[0], semk[0], semv[0]),
            (srci[1], dsti[1], edata[1], qrows[1], krows[1], vrows[1],
             semq[1], semk[1], semv[1]))

    def _prefetch(c, p):
        sr, ds_, ed, qr, kr, vr, sq, sk, sv = bufs[p]
        base = a0 + c * _CHUNK
        pltpu.sync_copy(src_hbm.at[pl.ds(base, _CHUNK)], sr)
        pltpu.sync_copy(dst_hbm.at[pl.ds(base, _CHUNK)], ds_)
        pltpu.sync_copy(edata_hbm.at[pl.ds(base * 4, 4 * _CHUNK + _L)], ed)
        pltpu.async_copy(q_hbm.at[ds_], qr, sq)
        pltpu.async_copy(k_hbm.at[sr], kr, sk)
        pltpu.async_copy(v_hbm.at[sr], vr, sv)

    def _wait(p):
        sr, ds_, ed, qr, kr, vr, sq, sk, sv = bufs[p]
        pltpu.make_async_copy(q_hbm.at[ds_], qr, sq).wait()
        pltpu.make_async_copy(k_hbm.at[sr], kr, sk).wait()
        pltpu.make_async_copy(v_hbm.at[sr], vr, sv).wait()

    def _make_edge_body(p):
        _, _, ed, qr, kr, vr, _, _, _ = bufs[p]

        def _edge_body(e, carry):
            m = carry[0]
            s = carry[1]
            cnt = carry[2]
            acc = carry[3:]
            row4 = ed[pl.ds(e * 4, _L)]
            node = row4[0]
            band = row4[1]
            attr = plsc.bitcast(row4, jnp.float32)[2]
            se = row4[3]
            row = jnp.full((_L,), e, jnp.int32)

            dot = zerof
            for d in range(_DK):
                col = iota16 + d
                qd = plsc.load_gather(qr, [row, col])
                kd = plsc.load_gather(kr, [row, col])
                dot = dot + qd * kd
            bias = plsc.load_gather(btbl_v, [jnp.full((_L,), band, jnp.int32),
                                             iota_h])
            a = (dot * 0.25 + bias) * attr
            mn = jnp.maximum(m, a)
            c = jnp.exp(m - mn)
            ev = jnp.exp(a - mn)
            s2 = s * c + ev
            acc2 = []
            for d in range(_DK):
                col = iota16 + d
                vd = plsc.load_gather(vr, [row, col])
                acc2.append(acc[d] * c + ev * vd)

            @pl.when(se == 1)
            def _():
                rcp = 1.0 / (s2 + 1e-16)
                cntv = jnp.full((_L,), cnt, jnp.int32)
                for d in range(_DK):
                    plsc.store_scatter(outbuf, [cntv, iota16 + d],
                                       acc2[d] * rcp, mask=lane_lt8)
                plsc.store_scatter(oidx, [cntv],
                                   jnp.full((_L,), node, jnp.int32),
                                   mask=lane0)

            cnt2 = cnt + se

            @pl.when(cnt2 == _FLUSH)
            def _():
                pltpu.async_copy(outbuf, out_hbm.at[oidx], semo).wait()
                for j in range(_FLUSH // _L):
                    oidx[pl.ds(j * _L, _L)] = padv

            cnt3 = jnp.where(cnt2 == _FLUSH, 0, cnt2)
            keep = se == 0
            m_o = jnp.where(keep, mn, neg_inf)
            s_o = jnp.where(keep, s2, zerof)
            acc_o = tuple(jnp.where(keep, a2, zerof) for a2 in acc2)
            return (m_o, s_o, cnt3) + acc_o
        return _edge_body

    edge_bodies = (_make_edge_body(0), _make_edge_body(1))

    @pl.when(nch > 0)
    def _():
        _prefetch(0, 0)

    def _pair_body(ci, carry):
        for b in (0, 1):
            c = ci * 2 + b

            @pl.when(c + 1 < nch)
            def _():
                _prefetch(c + 1, b ^ 1)

            @pl.when(c < nch)
            def _():
                _wait(b)

            base = a0 + c * _CHUNK
            lo = jnp.maximum(e_start - base, 0)
            hi = jnp.minimum(e_end - base, _CHUNK)
            carry = lax.fori_loop(lo, hi, edge_bodies[b], carry)
        return carry

    init = (neg_inf, zerof, jnp.int32(0)) + tuple(zerof for _ in range(_DK))
    carry = lax.fori_loop(0, (nch + 1) // 2, _pair_body, init)

    @pl.when(carry[2] > 0)
    def _():
        pltpu.async_copy(outbuf, out_hbm.at[oidx], semo).wait()


def _sc_edge(q, k, v, srcp, dstp, edata, meta, btbl):
    n = q.shape[0]
    mesh = plsc.VectorSubcoreMesh(core_axis_name="c", subcore_axis_name="s")
    f = pl.kernel(
        _sc_edge_body,
        out_type=jax.ShapeDtypeStruct((n + 8, _D), jnp.float32),
        mesh=mesh,
        compiler_params=pltpu.CompilerParams(use_tc_tiling_on_sc=False,
                                             needs_layout_passes=False),
        scratch_types=[
            pltpu.VMEM((4, _NW + _L), jnp.int32),
            pltpu.VMEM((6, _H), jnp.float32),
            [pltpu.VMEM((_CHUNK,), jnp.int32)] * 2,
            [pltpu.VMEM((_CHUNK,), jnp.int32)] * 2,
            [pltpu.VMEM((4 * _CHUNK + _L,), jnp.int32)] * 2,
            [pltpu.VMEM((_CHUNK, _D), jnp.float32)] * 2,
            [pltpu.VMEM((_CHUNK, _D), jnp.float32)] * 2,
            [pltpu.VMEM((_CHUNK, _D), jnp.float32)] * 2,
            pltpu.VMEM((_FLUSH, _D), jnp.float32),
            pltpu.VMEM((_FLUSH,), jnp.int32),
            pltpu.VMEM((32, _D), jnp.float32),
            [pltpu.SemaphoreType.DMA] * 2,
            [pltpu.SemaphoreType.DMA] * 2,
            [pltpu.SemaphoreType.DMA] * 2,
            pltpu.SemaphoreType.DMA,
        ],
    )
    return f(q, k, v, srcp, dstp, edata, meta, btbl)


def kernel(x, edge_index, edge_attr, band_ids, stage_ids, params):
    n = x.shape[0]
    e = edge_index.shape[1]

    src = edge_index[0].astype(jnp.int32)
    dst = edge_index[1].astype(jnp.int32)

    perm = jnp.argsort(dst)
    dst_s = dst[perm]
    src_s = src[perm]
    band_s = band_ids[perm].astype(jnp.int32)
    attr_s = edge_attr[perm]
    seg_end = jnp.concatenate(
        [(dst_s[1:] != dst_s[:-1]), jnp.ones((1,), jnp.bool_)]).astype(jnp.int32)

    tpos = (jnp.arange(1, _NW, dtype=jnp.int32) * e) // _NW
    node_at = dst_s[tpos]
    ecut_in = jnp.searchsorted(dst_s, node_at, side='left').astype(jnp.int32)
    e_cut = jnp.concatenate([jnp.zeros((1,), jnp.int32), ecut_in,
                             jnp.full((1,), e, jnp.int32)])
    ncut = jnp.concatenate([jnp.zeros((1,), jnp.int32), node_at,
                            jnp.full((1,), n, jnp.int32)])
    meta = jnp.stack([e_cut[:-1], e_cut[1:], ncut[:-1], ncut[1:]]).astype(jnp.int32)
    meta = jnp.pad(meta, ((0, 0), (0, _L)))

    pad = 2 * _CHUNK
    srcp = jnp.pad(src_s, (0, pad))
    dstp = jnp.pad(dst_s, (0, pad))
    attr_bits = jax.lax.bitcast_convert_type(attr_s, jnp.int32)
    edata = jnp.stack([dst_s, band_s, attr_bits, seg_end], axis=1).reshape(-1)
    edata = jnp.pad(edata, (0, 4 * pad))

    onehot = (stage_ids[:, None] == jnp.arange(_D)[None, :]).astype(jnp.float32)
    emb = jnp.zeros((_D, _D), jnp.float32).at[:params['stage_emb'].shape[0]].set(
        params['stage_emb'])

    h = _tc_in_proj(x, params['in_w'], params['in_b'].reshape(1, _D),
                    onehot, emb)

    for lp in params['layers']:
        q, k, v = _tc_qkv(h, lp['wq'], lp['wk'], lp['wv'])
        btbl = jnp.zeros((6, _H), jnp.float32).at[:5].set(lp['band_bias'])
        agg = _sc_edge(q, k, v, srcp, dstp, edata, meta, btbl)
        h = _tc_node(agg, h, lp)

    return _tc_out(h, params['out_w'], params['out_b'])


# speedup vs baseline: 44.7541x; 2.1998x over previous
import functools
import math

import jax
import jax.numpy as jnp
from jax import lax
from jax.experimental import pallas as pl
from jax.experimental.pallas import tpu as pltpu
from jax.experimental.pallas import tpu_sc as plsc

_NC = 2
_NS = 16
_NW = _NC * _NS
_L = 16
_CHUNK = 128

_H = 8
_DK = 16
_D = 128


def _dot(a, b):
    return jax.lax.dot_general(a, b, (((1,), (0,)), ((), ())),
                               precision=jax.lax.Precision.HIGHEST,
                               preferred_element_type=jnp.float32)


def _ln(x, g, b, eps=1e-5):
    mu = jnp.mean(x, axis=-1, keepdims=True)
    var = jnp.mean((x - mu) ** 2, axis=-1, keepdims=True)
    return (x - mu) / jnp.sqrt(var + eps) * g + b


def _erf(x):
    s = jnp.sign(x)
    ax = jnp.abs(x)
    t = 1.0 / (1.0 + 0.3275911 * ax)
    poly = t * (0.254829592 + t * (-0.284496736 + t * (1.421413741
               + t * (-1.453152027 + t * 1.061405429))))
    return s * (1.0 - poly * jnp.exp(-ax * ax))


def _gelu(x):
    return 0.5 * x * (1.0 + _erf(x * (1.0 / math.sqrt(2.0))))


def _in_proj_body(x_ref, w_ref, b_ref, oh_ref, emb_ref, o_ref):
    o_ref[...] = (_dot(x_ref[...], w_ref[...]) + b_ref[...]
                  + _dot(oh_ref[...], emb_ref[...]))


def _qkv_body(h_ref, wq_ref, wk_ref, wv_ref, q_ref, k_ref, v_ref):
    h = h_ref[...]
    q_ref[...] = _dot(h, wq_ref[...])
    k_ref[...] = _dot(h, wk_ref[...])
    v_ref[...] = _dot(h, wv_ref[...])


def _node_body(agg_ref, h_ref, wo_ref, bo_ref, g1_ref, b1n_ref,
               w1_ref, b1_ref, w2_ref, b2_ref, g2_ref, b2n_ref, o_ref):
    u = _dot(agg_ref[...], wo_ref[...]) + bo_ref[...] + h_ref[...]
    u = _ln(u, g1_ref[...], b1n_ref[...])
    ff = _dot(_gelu(_dot(u, w1_ref[...]) + b1_ref[...]), w2_ref[...]) + b2_ref[...]
    o_ref[...] = _ln(u + ff, g2_ref[...], b2n_ref[...])


def _out_body(h_ref, w_ref, b_ref, o_ref):
    o_ref[...] = _dot(h_ref[...], w_ref[...]) + b_ref[...]


def _row_spec(blk, d):
    return pl.BlockSpec((blk, d), lambda i: (i, 0))


def _full_spec(a, b):
    return pl.BlockSpec((a, b), lambda i: (0, 0))


def _tc_in_proj(x, w, b, onehot, emb):
    n, d_in = x.shape
    blk = 1000
    return pl.pallas_call(
        _in_proj_body,
        grid=(n // blk,),
        in_specs=[_row_spec(blk, d_in), _full_spec(d_in, _D),
                  _full_spec(1, _D), _row_spec(blk, _D), _full_spec(_D, _D)],
        out_specs=_row_spec(blk, _D),
        out_shape=jax.ShapeDtypeStruct((n, _D), jnp.float32),
    )(x, w, b, onehot, emb)


def _tc_qkv(h, wq, wk, wv):
    n = h.shape[0]
    blk = 1000
    sds = jax.ShapeDtypeStruct((n, _D), jnp.float32)
    return pl.pallas_call(
        _qkv_body,
        grid=(n // blk,),
        in_specs=[_row_spec(blk, _D)] + [_full_spec(_D, _D)] * 3,
        out_specs=[_row_spec(blk, _D)] * 3,
        out_shape=[sds, sds, sds],
    )(h, wq, wk, wv)


def _tc_node(agg, h, lp):
    n = h.shape[0]
    blk = 1000
    return pl.pallas_call(
        _node_body,
        grid=(n // blk,),
        in_specs=[_row_spec(blk, _D), _row_spec(blk, _D),
                  _full_spec(_D, _D), _full_spec(1, _D),
                  _full_spec(1, _D), _full_spec(1, _D),
                  _full_spec(_D, 4 * _D), _full_spec(1, 4 * _D),
                  _full_spec(4 * _D, _D), _full_spec(1, _D),
                  _full_spec(1, _D), _full_spec(1, _D)],
        out_specs=_row_spec(blk, _D),
        out_shape=jax.ShapeDtypeStruct((n, _D), jnp.float32),
    )(agg, h, lp['wo'], lp['bo'].reshape(1, _D),
      lp['ln1_g'].reshape(1, _D), lp['ln1_b'].reshape(1, _D),
      lp['w1'], lp['b1'].reshape(1, 4 * _D), lp['w2'],
      lp['b2'].reshape(1, _D), lp['ln2_g'].reshape(1, _D),
      lp['ln2_b'].reshape(1, _D))


def _tc_out(h, w, b):
    n = h.shape[0]
    blk = 1000
    return pl.pallas_call(
        _out_body,
        grid=(n // blk,),
        in_specs=[_row_spec(blk, _D), _full_spec(_D, _D), _full_spec(1, _D)],
        out_specs=_row_spec(blk, _D),
        out_shape=jax.ShapeDtypeStruct((n, _D), jnp.float32),
    )(h, w, b.reshape(1, _D))


_FLUSH = 64


def _sc_edge_body(q_hbm, k_hbm, v_hbm, src_hbm, dst_hbm, edata_hbm,
                  meta_hbm, btbl_hbm, out_hbm,
                  meta_v, btbl_v, srci, dsti, edata, qrows, krows, vrows,
                  outbuf, oidx, zbuf, semq, semk, semv, semo):
    wid = lax.axis_index("s") * _NC + lax.axis_index("c")
    pad_row = out_hbm.shape[0] - 1

    pltpu.sync_copy(meta_hbm, meta_v)
    pltpu.sync_copy(btbl_hbm, btbl_v)

    e_start = meta_v[0, pl.ds(wid, _L)][0]
    e_end = meta_v[1, pl.ds(wid, _L)][0]
    n_start = meta_v[2, pl.ds(wid, _L)][0]
    n_end = meta_v[3, pl.ds(wid, _L)][0]

    iota = lax.iota(jnp.int32, _L)
    iota16 = iota * _DK
    iota_h = jnp.minimum(iota, _H - 1)
    lane_lt8 = iota < _H
    lane0 = iota < 1
    neg_inf = jnp.full((_L,), -jnp.inf, jnp.float32)
    zerof = jnp.zeros((_L,), jnp.float32)
    padv = jnp.full((_L,), pad_row, jnp.int32)

    zb = zbuf.shape[0]
    def _zb(i, _):
        for j in range(_D // _L):
            zbuf[i, pl.ds(j * _L, _L)] = zerof
        return 0
    lax.fori_loop(0, zb, _zb, 0)
    for j in range(_FLUSH // _L):
        oidx[pl.ds(j * _L, _L)] = padv

    ncnt = n_end - n_start
    nblocks = ncnt // zb

    def _zero_block(i, _):
        pltpu.sync_copy(zbuf, out_hbm.at[pl.ds(n_start + i * zb, zb)])
        return 0
    lax.fori_loop(0, nblocks, _zero_block, 0)

    @pl.when(ncnt >= zb)
    def _():
        pltpu.sync_copy(zbuf, out_hbm.at[pl.ds(n_end - zb, zb)])

    @pl.when(jnp.logical_and(ncnt < zb, ncnt > 0))
    def _():
        def _zr(i, _):
            pltpu.sync_copy(zbuf.at[pl.ds(0, 1)],
                            out_hbm.at[pl.ds(n_start + i, 1)])
            return 0
        lax.fori_loop(0, ncnt, _zr, 0)

    a0 = (e_start // 8) * 8
    nch = (e_end - a0 + _CHUNK - 1) // _CHUNK

    bufs = ((srci[0], dsti[0], edata[0], qrows[0], krows[0], vrows[0],
             semq[0], semk[0], semv[0]),
            (srci[1], dsti[1], edata[1], qrows[1], krows[1], vrows[1],
             semq[1], semk[1], semv[1]))

    def _prefetch(c, p):
        sr, ds_, ed, qr, kr, vr, sq, sk, sv = bufs[p]
        base = a0 + c * _CHUNK
        pltpu.sync_copy(src_hbm.at[pl.ds(base, _CHUNK)], sr)
        pltpu.sync_copy(dst_hbm.at[pl.ds(base, _CHUNK)], ds_)
        pltpu.sync_copy(edata_hbm.at[pl.ds(base * 4, 4 * _CHUNK + _L)], ed)
        pltpu.async_copy(q_hbm.at[ds_], qr, sq)
        pltpu.async_copy(k_hbm.at[sr], kr, sk)
        pltpu.async_copy(v_hbm.at[sr], vr, sv)

    def _wait(p):
        sr, ds_, ed, qr, kr, vr, sq, sk, sv = bufs[p]
        pltpu.make_async_copy(q_hbm.at[ds_], qr, sq).wait()
        pltpu.make_async_copy(k_hbm.at[sr], kr, sk).wait()
        pltpu.make_async_copy(v_hbm.at[sr], vr, sv).wait()

    sh8 = (iota + 8) & 15
    h8 = iota & 7
    offb = (iota & 7) * _DK + (iota >= _H).astype(jnp.int32)

    def _take(x, idx):
        return x.at[idx].get(mode='promise_in_bounds')

    def _make_edge_body(p):
        _, _, ed, qr, kr, vr, _, _, _ = bufs[p]

        def _edge_body(e, carry):
            m = carry[0]
            s = carry[1]
            cnt = carry[2]
            acc = carry[3:]
            row4 = ed[pl.ds(e * 4, _L)]
            node = row4[0]
            band = row4[1]
            attr = plsc.bitcast(row4, jnp.float32)[2]
            se = row4[3]

            dot0 = zerof
            dot1 = zerof
            for j in range(_H):
                qd = qr[e, pl.ds(j * _L, _L)]
                kd = kr[e, pl.ds(j * _L, _L)]
                if j % 2 == 0:
                    dot0 = dot0 + qd * kd
                else:
                    dot1 = dot1 + qd * kd
            dot = dot0 + dot1
            dot = dot + _take(dot, sh8)
            bias = btbl_v[band, pl.ds(0, _L)]
            a = (dot * 0.25 + bias) * attr
            mn = jnp.maximum(m, a)
            c = jnp.exp(m - mn)
            ev = jnp.exp(a - mn)
            s2 = s * c + ev
            c2 = _take(c, h8)
            ev2 = _take(ev, h8)
            acc2 = []
            for j in range(_H):
                vd = vr[e, pl.ds(j * _L, _L)]
                acc2.append(acc[j] * c2 + ev2 * vd)

            @pl.when(se == 1)
            def _():
                rcp2 = _take(1.0 / (s2 + 1e-16), h8)
                cntv = jnp.full((_L,), cnt, jnp.int32)
                for j in range(_H):
                    plsc.store_scatter(outbuf, [cntv, offb + 2 * j],
                                       acc2[j] * rcp2)
                plsc.store_scatter(oidx, [cntv],
                                   jnp.full((_L,), node, jnp.int32),
                                   mask=lane0)

            cnt2 = cnt + se

            @pl.when(cnt2 == _FLUSH)
            def _():
                pltpu.async_copy(outbuf, out_hbm.at[oidx], semo).wait()
                for j in range(_FLUSH // _L):
                    oidx[pl.ds(j * _L, _L)] = padv

            cnt3 = jnp.where(cnt2 == _FLUSH, 0, cnt2)
            keep = se == 0
            m_o = jnp.where(keep, mn, neg_inf)
            s_o = jnp.where(keep, s2, zerof)
            acc_o = tuple(jnp.where(keep, a2, zerof) for a2 in acc2)
            return (m_o, s_o, cnt3) + acc_o
        return _edge_body

    edge_bodies = (_make_edge_body(0), _make_edge_body(1))

    @pl.when(nch > 0)
    def _():
        _prefetch(0, 0)

    def _pair_body(ci, carry):
        for b in (0, 1):
            c = ci * 2 + b

            @pl.when(c + 1 < nch)
            def _():
                _prefetch(c + 1, b ^ 1)

            @pl.when(c < nch)
            def _():
                _wait(b)

            base = a0 + c * _CHUNK
            lo = jnp.maximum(e_start - base, 0)
            hi = jnp.minimum(e_end - base, _CHUNK)
            carry = lax.fori_loop(lo, hi, edge_bodies[b], carry)
        return carry

    init = (neg_inf, zerof, jnp.int32(0)) + tuple(zerof for _ in range(_H))
    carry = lax.fori_loop(0, (nch + 1) // 2, _pair_body, init)

    @pl.when(carry[2] > 0)
    def _():
        pltpu.async_copy(outbuf, out_hbm.at[oidx], semo).wait()


def _sc_edge(q, k, v, srcp, dstp, edata, meta, btbl):
    n = q.shape[0]
    mesh = plsc.VectorSubcoreMesh(core_axis_name="c", subcore_axis_name="s")
    f = pl.kernel(
        _sc_edge_body,
        out_type=jax.ShapeDtypeStruct((n + 8, _D), jnp.float32),
        mesh=mesh,
        compiler_params=pltpu.CompilerParams(use_tc_tiling_on_sc=False,
                                             needs_layout_passes=False),
        scratch_types=[
            pltpu.VMEM((4, _NW + _L), jnp.int32),
            pltpu.VMEM((6, _L), jnp.float32),
            [pltpu.VMEM((_CHUNK,), jnp.int32)] * 2,
            [pltpu.VMEM((_CHUNK,), jnp.int32)] * 2,
            [pltpu.VMEM((4 * _CHUNK + _L,), jnp.int32)] * 2,
            [pltpu.VMEM((_CHUNK, _D), jnp.float32)] * 2,
            [pltpu.VMEM((_CHUNK, _D), jnp.float32)] * 2,
            [pltpu.VMEM((_CHUNK, _D), jnp.float32)] * 2,
            pltpu.VMEM((_FLUSH, _D), jnp.float32),
            pltpu.VMEM((_FLUSH,), jnp.int32),
            pltpu.VMEM((32, _D), jnp.float32),
            [pltpu.SemaphoreType.DMA] * 2,
            [pltpu.SemaphoreType.DMA] * 2,
            [pltpu.SemaphoreType.DMA] * 2,
            pltpu.SemaphoreType.DMA,
        ],
    )
    return f(q, k, v, srcp, dstp, edata, meta, btbl)


def kernel(x, edge_index, edge_attr, band_ids, stage_ids, params):
    n = x.shape[0]
    e = edge_index.shape[1]

    src = edge_index[0].astype(jnp.int32)
    dst = edge_index[1].astype(jnp.int32)

    perm = jnp.argsort(dst)
    dst_s = dst[perm]
    src_s = src[perm]
    band_s = band_ids[perm].astype(jnp.int32)
    attr_s = edge_attr[perm]
    seg_end = jnp.concatenate(
        [(dst_s[1:] != dst_s[:-1]), jnp.ones((1,), jnp.bool_)]).astype(jnp.int32)

    tpos = (jnp.arange(1, _NW, dtype=jnp.int32) * e) // _NW
    node_at = dst_s[tpos]
    ecut_in = jnp.searchsorted(dst_s, node_at, side='left').astype(jnp.int32)
    e_cut = jnp.concatenate([jnp.zeros((1,), jnp.int32), ecut_in,
                             jnp.full((1,), e, jnp.int32)])
    ncut = jnp.concatenate([jnp.zeros((1,), jnp.int32), node_at,
                            jnp.full((1,), n, jnp.int32)])
    meta = jnp.stack([e_cut[:-1], e_cut[1:], ncut[:-1], ncut[1:]]).astype(jnp.int32)
    meta = jnp.pad(meta, ((0, 0), (0, _L)))

    pad = 2 * _CHUNK
    srcp = jnp.pad(src_s, (0, pad))
    dstp = jnp.pad(dst_s, (0, pad))
    attr_bits = jax.lax.bitcast_convert_type(attr_s, jnp.int32)
    edata = jnp.stack([dst_s, band_s, attr_bits, seg_end], axis=1).reshape(-1)
    edata = jnp.pad(edata, (0, 4 * pad))

    cc = jnp.arange(_D, dtype=jnp.int32)
    tcols = (cc % _H) * _DK + cc // _H

    onehot = (stage_ids[:, None] == jnp.arange(_D)[None, :]).astype(jnp.float32)
    emb = jnp.zeros((_D, _D), jnp.float32).at[:params['stage_emb'].shape[0]].set(
        params['stage_emb'])

    h = _tc_in_proj(x, params['in_w'], params['in_b'].reshape(1, _D),
                    onehot, emb)

    for lp in params['layers']:
        q, k, v = _tc_qkv(h, lp['wq'][:, tcols], lp['wk'][:, tcols],
                          lp['wv'][:, tcols])
        btbl = jnp.zeros((6, _L), jnp.float32).at[:5, :_H].set(lp['band_bias'])
        agg = _sc_edge(q, k, v, srcp, dstp, edata, meta, btbl)
        h = _tc_node(agg, h, lp)

    return _tc_out(h, params['out_w'], params['out_b'])
